# Initial kernel scaffold; baseline (speedup 1.0000x reference)
import jax
import jax.numpy as jnp
from jax.experimental import pallas as pl


def kernel(loc_data, conf_data, priors, ground_truth):
    raise NotImplementedError("write your pallas kernel here")


# speedup vs baseline: 2.2344x; 2.2344x over previous
import jax
import jax.numpy as jnp
from jax import lax
from jax.experimental import pallas as pl
from jax.experimental.pallas import tpu as pltpu

_P = 131072
_G = 50
_C = 81
_BA = 1024
_BC = 2048


def _match_body(pf_ref, tt_ref, bto_ref, bti_ref, bpv_ref, bpi_ref):
    blk = pl.program_id(0)
    pf = pf_ref[...]
    tt = tt_ref[...]
    px1, py1, px2, py2 = pf[:, 0:1], pf[:, 1:2], pf[:, 2:3], pf[:, 3:4]
    tx1, ty1, tx2, ty2 = tt[0:1, :], tt[1:2, :], tt[2:3, :], tt[3:4, :]
    w = jnp.maximum(jnp.minimum(px2, tx2) - jnp.maximum(px1, tx1), 0.0)
    h = jnp.maximum(jnp.minimum(py2, ty2) - jnp.maximum(py1, ty1), 0.0)
    inter = w * h
    aa = (px2 - px1) * (py2 - py1)
    ab = (tx2 - tx1) * (ty2 - ty1)
    iou = inter / (aa + ab - inter)

    g_iota = lax.broadcasted_iota(jnp.int32, (_BA, _G), 1)
    r_iota = lax.broadcasted_iota(jnp.int32, (_BA, _G), 0)
    bto = jnp.max(iou, axis=1)
    bti = jnp.min(jnp.where(iou == bto[:, None], g_iota, _G), axis=1)
    bto_ref[...] = bto
    bti_ref[...] = bti

    colmax = jnp.max(iou, axis=0, keepdims=True)
    colidx = jnp.min(jnp.where(iou == colmax, r_iota, _BA),
                     axis=0, keepdims=True) + blk * _BA

    @pl.when(blk == 0)
    def _():
        bpv_ref[...] = jnp.full((1, _G), -1.0, jnp.float32)
        bpi_ref[...] = jnp.zeros((1, _G), jnp.int32)

    acc_v = bpv_ref[...]
    upd = colmax > acc_v
    bpv_ref[...] = jnp.where(upd, colmax, acc_v)
    bpi_ref[...] = jnp.where(upd, colidx, bpi_ref[...])


def _targets_body(bto_ref, bti_ref, bpi_ref, tt_ref, lab_ref, pcf_ref,
                  loc_ref, ct_ref, tgt_ref, locl_ref, npos_ref):
    blk = pl.program_id(0)
    bto = bto_ref[...][:, None]
    bti = bti_ref[...][:, None]
    bpi = bpi_ref[...]
    gp = blk * _BA + lax.broadcasted_iota(jnp.int32, (_BA, 1), 0)
    g_iota = lax.broadcasted_iota(jnp.int32, (_BA, _G), 1)
    forced_g = jnp.max(jnp.where(bpi == gp, g_iota, -1), axis=1,
                       keepdims=True)
    forced = forced_g >= 0
    bti2 = jnp.where(forced, forced_g, bti)
    bto2 = jnp.where(forced, 2.0, bto)
    onehot = g_iota == bti2

    lab = lab_ref[...]
    labm = jnp.sum(jnp.where(onehot, lab, 0.0), axis=1, keepdims=True)
    conf_t = jnp.where(bto2 >= 0.5, labm.astype(jnp.int32) + 1, 0)

    tt = tt_ref[...]
    mx1 = jnp.sum(jnp.where(onehot, tt[0:1, :], 0.0), axis=1, keepdims=True)
    my1 = jnp.sum(jnp.where(onehot, tt[1:2, :], 0.0), axis=1, keepdims=True)
    mx2 = jnp.sum(jnp.where(onehot, tt[2:3, :], 0.0), axis=1, keepdims=True)
    my2 = jnp.sum(jnp.where(onehot, tt[3:4, :], 0.0), axis=1, keepdims=True)

    pcf = pcf_ref[...]
    cx, cy, w, h = pcf[:, 0:1], pcf[:, 1:2], pcf[:, 2:3], pcf[:, 3:4]
    gcx = ((mx1 + mx2) * 0.5 - cx) / (0.1 * w)
    gcy = ((my1 + my2) * 0.5 - cy) / (0.1 * h)
    gw = jnp.log((mx2 - mx1) / w) / 0.2
    gh = jnp.log((my2 - my1) / h) / 0.2

    loc = loc_ref[...]

    def sl1(d):
        a = jnp.abs(d)
        return jnp.where(a < 1.0, 0.5 * d * d, a - 0.5)

    l = (sl1(loc[:, 0:1] - gcx) + sl1(loc[:, 1:2] - gcy)
         + sl1(loc[:, 2:3] - gw) + sl1(loc[:, 3:4] - gh))
    posf = (conf_t > 0).astype(jnp.float32)
    ct_ref[...] = conf_t[:, 0]
    tgt_ref[...] = gp[:, 0] * _C + conf_t[:, 0]

    @pl.when(blk == 0)
    def _():
        locl_ref[...] = jnp.zeros((1, 1), jnp.float32)
        npos_ref[...] = jnp.zeros((1, 1), jnp.float32)

    locl_ref[...] = locl_ref[...] + jnp.sum(l * posf)
    npos_ref[...] = npos_ref[...] + jnp.sum(posf)


def _conf_body(x_ref, ct_ref, ce_ref):
    x = x_ref[...]
    ct = ct_ref[...][:, None]
    rmax = jnp.max(x, axis=1, keepdims=True)
    s = jnp.sum(jnp.exp(x - rmax), axis=1, keepdims=True)
    lse = jnp.log(s) + rmax
    cidx = lax.broadcasted_iota(jnp.int32, (_BC, _C), 1)
    xt = jnp.sum(jnp.where(cidx == ct, x, 0.0), axis=1, keepdims=True)
    ce_ref[...] = (lse - xt)[:, 0]


def _final_body(ce_ref, ct_ref, locl_ref, npos_ref, out_ref):
    ce = ce_ref[...]
    pos = ct_ref[...] > 0
    posf = pos.astype(jnp.float32)
    ce_pos = jnp.sum(ce * posf)
    cl = jnp.where(pos, 0.0, ce)
    bits = lax.bitcast_convert_type(cl, jnp.int32)
    npos = npos_ref[0, 0]
    k = jnp.minimum(3 * npos.astype(jnp.int32), _P - 1)
    kk = k + 1

    def body(i, acc):
        cand = acc | (jnp.int32(1) << (30 - i))
        cnt = jnp.sum((bits >= cand).astype(jnp.int32))
        return jnp.where(cnt >= kk, cand, acc)

    pivot = lax.bitcast_convert_type(lax.fori_loop(0, 31, body, jnp.int32(0)),
                                     jnp.float32)
    negf = (cl > pivot).astype(jnp.float32)
    ce_neg = jnp.sum(ce * negf)
    loss = (locl_ref[0, 0] + ce_pos + ce_neg) / npos
    out_ref[...] = jnp.zeros((1, 1), jnp.float32) + loss


def kernel(loc_data, conf_data, priors, ground_truth):
    pf = priors[0].reshape(_P, 4)
    pcf = priors[1].reshape(_P, 4)
    tt = ground_truth[:, :4].T
    lab = ground_truth[:, 4].reshape(1, _G)
    conf_flat = conf_data.reshape(_P, _C)
    loc_flat = loc_data.reshape(_P, 4)
    nba = _P // _BA
    nbc = _P // _BC

    bto, bti, _, bpi = pl.pallas_call(
        _match_body,
        grid=(nba,),
        in_specs=[
            pl.BlockSpec((_BA, 4), lambda i: (i, 0)),
            pl.BlockSpec((4, _G), lambda i: (0, 0)),
        ],
        out_specs=[
            pl.BlockSpec((_BA,), lambda i: (i,)),
            pl.BlockSpec((_BA,), lambda i: (i,)),
            pl.BlockSpec((1, _G), lambda i: (0, 0)),
            pl.BlockSpec((1, _G), lambda i: (0, 0)),
        ],
        out_shape=[
            jax.ShapeDtypeStruct((_P,), jnp.float32),
            jax.ShapeDtypeStruct((_P,), jnp.int32),
            jax.ShapeDtypeStruct((1, _G), jnp.float32),
            jax.ShapeDtypeStruct((1, _G), jnp.int32),
        ],
    )(pf, tt)

    conf_t, tgt_idx, locl, npos = pl.pallas_call(
        _targets_body,
        grid=(nba,),
        in_specs=[
            pl.BlockSpec((_BA,), lambda i: (i,)),
            pl.BlockSpec((_BA,), lambda i: (i,)),
            pl.BlockSpec((1, _G), lambda i: (0, 0)),
            pl.BlockSpec((4, _G), lambda i: (0, 0)),
            pl.BlockSpec((1, _G), lambda i: (0, 0)),
            pl.BlockSpec((_BA, 4), lambda i: (i, 0)),
            pl.BlockSpec((_BA, 4), lambda i: (i, 0)),
        ],
        out_specs=[
            pl.BlockSpec((_BA,), lambda i: (i,)),
            pl.BlockSpec((_BA,), lambda i: (i,)),
            pl.BlockSpec((1, 1), lambda i: (0, 0)),
            pl.BlockSpec((1, 1), lambda i: (0, 0)),
        ],
        out_shape=[
            jax.ShapeDtypeStruct((_P,), jnp.int32),
            jax.ShapeDtypeStruct((_P,), jnp.int32),
            jax.ShapeDtypeStruct((1, 1), jnp.float32),
            jax.ShapeDtypeStruct((1, 1), jnp.float32),
        ],
    )(bto, bti, bpi, tt, lab, pcf, loc_flat)

    ce = pl.pallas_call(
        _conf_body,
        grid=(nbc,),
        in_specs=[
            pl.BlockSpec((_BC, _C), lambda i: (i, 0)),
            pl.BlockSpec((_BC,), lambda i: (i,)),
        ],
        out_specs=pl.BlockSpec((_BC,), lambda i: (i,)),
        out_shape=jax.ShapeDtypeStruct((_P,), jnp.float32),
    )(conf_flat, conf_t)

    out = pl.pallas_call(
        _final_body,
        in_specs=[
            pl.BlockSpec((_P // 128, 128), lambda: (0, 0)),
            pl.BlockSpec((_P // 128, 128), lambda: (0, 0)),
            pl.BlockSpec((1, 1), lambda: (0, 0)),
            pl.BlockSpec((1, 1), lambda: (0, 0)),
        ],
        out_specs=pl.BlockSpec((1, 1), lambda: (0, 0)),
        out_shape=jax.ShapeDtypeStruct((1, 1), jnp.float32),
    )(ce.reshape(_P // 128, 128), conf_t.reshape(_P // 128, 128), locl, npos)

    return out[0, 0]


# speedup vs baseline: 4.1349x; 1.8506x over previous
import jax
import jax.numpy as jnp
from jax import lax
from jax.experimental import pallas as pl
from jax.experimental.pallas import tpu as pltpu

_P = 131072
_G = 50
_C = 81
_R = _P // 128
_RA = 8
_BC = 2048
_NBA = _R // _RA
_NBC = _P // _BC


def _match_body(pft_ref, tt_ref, bto_ref, bti_ref, accv_ref, acci_ref,
                bpi_ref):
    blk = pl.program_id(0)
    pft = pft_ref[...]
    tt = tt_ref[...]
    px1, py1 = pft[0][None], pft[1][None]
    px2, py2 = pft[2][None], pft[3][None]
    tx1, ty1, tx2, ty2 = tt[0], tt[1], tt[2], tt[3]
    w = jnp.maximum(jnp.minimum(px2, tx2) - jnp.maximum(px1, tx1), 0.0)
    h = jnp.maximum(jnp.minimum(py2, ty2) - jnp.maximum(py1, ty1), 0.0)
    inter = w * h
    aa = (px2 - px1) * (py2 - py1)
    ab = (tx2 - tx1) * (ty2 - ty1)
    iou = inter / (aa + ab - inter)

    g_iota = lax.broadcasted_iota(jnp.int32, (_G, _RA, 128), 0)
    bto = jnp.max(iou, axis=0)
    bti = jnp.min(jnp.where(iou == bto[None], g_iota, _G), axis=0)
    bto_ref[...] = bto
    bti_ref[...] = bti

    gp = (blk * (_RA * 128)
          + lax.broadcasted_iota(jnp.int32, (_RA, 128), 0) * 128
          + lax.broadcasted_iota(jnp.int32, (_RA, 128), 1))[None]

    @pl.when(blk == 0)
    def _():
        accv_ref[...] = jnp.full((_G, _RA, 128), -1.0, jnp.float32)
        acci_ref[...] = jnp.zeros((_G, _RA, 128), jnp.int32)

    acc_v = accv_ref[...]
    upd = iou > acc_v
    accv_ref[...] = jnp.where(upd, iou, acc_v)
    acci_ref[...] = jnp.where(upd, gp, acci_ref[...])

    @pl.when(blk == _NBA - 1)
    def _():
        av = accv_ref[...]
        vmax = jnp.max(av, axis=(1, 2), keepdims=True)
        bpi_ref[...] = jnp.min(
            jnp.where(av == vmax, acci_ref[...], _P), axis=(1, 2),
            keepdims=True)


def _targets_body(bto_ref, bti_ref, bpi_ref, tt_ref, lab_ref, pcft_ref,
                  loct_ref, ct_ref, locl_ref, npos_ref):
    blk = pl.program_id(0)
    bto = bto_ref[...]
    bti = bti_ref[...]
    bpi = bpi_ref[...]
    gp = (blk * (_RA * 128)
          + lax.broadcasted_iota(jnp.int32, (_RA, 128), 0) * 128
          + lax.broadcasted_iota(jnp.int32, (_RA, 128), 1))
    g_iota = lax.broadcasted_iota(jnp.int32, (_G, _RA, 128), 0)
    forced_g = jnp.max(jnp.where(bpi == gp[None], g_iota, -1), axis=0)
    forced = forced_g >= 0
    bti2 = jnp.where(forced, forced_g, bti)
    bto2 = jnp.where(forced, 2.0, bto)
    onehot = g_iota == bti2[None]

    lab = lab_ref[...]
    labm = jnp.sum(jnp.where(onehot, lab, 0.0), axis=0)
    conf_t = jnp.where(bto2 >= 0.5, labm.astype(jnp.int32) + 1, 0)

    tt = tt_ref[...]
    mx1 = jnp.sum(jnp.where(onehot, tt[0], 0.0), axis=0)
    my1 = jnp.sum(jnp.where(onehot, tt[1], 0.0), axis=0)
    mx2 = jnp.sum(jnp.where(onehot, tt[2], 0.0), axis=0)
    my2 = jnp.sum(jnp.where(onehot, tt[3], 0.0), axis=0)

    pcf = pcft_ref[...]
    cx, cy, w, h = pcf[0], pcf[1], pcf[2], pcf[3]
    gcx = ((mx1 + mx2) * 0.5 - cx) / (0.1 * w)
    gcy = ((my1 + my2) * 0.5 - cy) / (0.1 * h)
    gw = jnp.log((mx2 - mx1) / w) / 0.2
    gh = jnp.log((my2 - my1) / h) / 0.2

    loc = loct_ref[...]

    def sl1(d):
        a = jnp.abs(d)
        return jnp.where(a < 1.0, 0.5 * d * d, a - 0.5)

    l = sl1(loc[0] - gcx) + sl1(loc[1] - gcy) + sl1(loc[2] - gw) \
        + sl1(loc[3] - gh)
    posf = (conf_t > 0).astype(jnp.float32)
    ct_ref[...] = conf_t.reshape(_RA * 128)

    @pl.when(blk == 0)
    def _():
        locl_ref[...] = jnp.zeros((1, 1), jnp.float32)
        npos_ref[...] = jnp.zeros((1, 1), jnp.float32)

    locl_ref[...] = locl_ref[...] + jnp.sum(l * posf)
    npos_ref[...] = npos_ref[...] + jnp.sum(posf)


def _conf_body(x_ref, ct_ref, ce_ref):
    x = x_ref[...]
    ct = ct_ref[...][:, None]
    rmax = jnp.max(x, axis=1, keepdims=True)
    s = jnp.sum(jnp.exp(x - rmax), axis=1, keepdims=True)
    lse = jnp.log(s) + rmax
    cidx = lax.broadcasted_iota(jnp.int32, (_BC, _C), 1)
    xt = jnp.sum(jnp.where(cidx == ct, x, 0.0), axis=1, keepdims=True)
    ce_ref[...] = (lse - xt)[:, 0]


def _final_body(ce_ref, ct_ref, locl_ref, npos_ref, out_ref):
    ce = ce_ref[...]
    pos = ct_ref[...] > 0
    posf = pos.astype(jnp.float32)
    ce_pos = jnp.sum(ce * posf)
    cl = jnp.where(pos, 0.0, ce)
    bits = lax.bitcast_convert_type(cl, jnp.int32)
    npos = npos_ref[0, 0]
    k = jnp.minimum(3 * npos.astype(jnp.int32), _P - 1)
    kk = k + 1

    def body(i, acc):
        cand = acc | (jnp.int32(1) << (30 - i))
        cnt = jnp.sum((bits >= cand).astype(jnp.int32))
        return jnp.where(cnt >= kk, cand, acc)

    pivot = lax.bitcast_convert_type(lax.fori_loop(0, 31, body, jnp.int32(0)),
                                     jnp.float32)
    negf = (cl > pivot).astype(jnp.float32)
    ce_neg = jnp.sum(ce * negf)
    loss = (locl_ref[0, 0] + ce_pos + ce_neg) / npos
    out_ref[...] = jnp.zeros((1, 1), jnp.float32) + loss


def kernel(loc_data, conf_data, priors, ground_truth):
    pft = priors[0].reshape(_P, 4).T.reshape(4, _R, 128)
    pcft = priors[1].reshape(_P, 4).T.reshape(4, _R, 128)
    tt = ground_truth[:, :4].T.reshape(4, _G, 1, 1)
    lab = ground_truth[:, 4].reshape(_G, 1, 1)
    conf_flat = conf_data.reshape(_P, _C)
    loct = loc_data.reshape(_P, 4).T.reshape(4, _R, 128)

    bto, bti, _, _, bpi = pl.pallas_call(
        _match_body,
        grid=(_NBA,),
        in_specs=[
            pl.BlockSpec((4, _RA, 128), lambda i: (0, i, 0)),
            pl.BlockSpec((4, _G, 1, 1), lambda i: (0, 0, 0, 0)),
        ],
        out_specs=[
            pl.BlockSpec((_RA, 128), lambda i: (i, 0)),
            pl.BlockSpec((_RA, 128), lambda i: (i, 0)),
            pl.BlockSpec((_G, _RA, 128), lambda i: (0, 0, 0)),
            pl.BlockSpec((_G, _RA, 128), lambda i: (0, 0, 0)),
            pl.BlockSpec((_G, 1, 1), lambda i: (0, 0, 0)),
        ],
        out_shape=[
            jax.ShapeDtypeStruct((_R, 128), jnp.float32),
            jax.ShapeDtypeStruct((_R, 128), jnp.int32),
            jax.ShapeDtypeStruct((_G, _RA, 128), jnp.float32),
            jax.ShapeDtypeStruct((_G, _RA, 128), jnp.int32),
            jax.ShapeDtypeStruct((_G, 1, 1), jnp.int32),
        ],
    )(pft, tt)

    conf_t, locl, npos = pl.pallas_call(
        _targets_body,
        grid=(_NBA,),
        in_specs=[
            pl.BlockSpec((_RA, 128), lambda i: (i, 0)),
            pl.BlockSpec((_RA, 128), lambda i: (i, 0)),
            pl.BlockSpec((_G, 1, 1), lambda i: (0, 0, 0)),
            pl.BlockSpec((4, _G, 1, 1), lambda i: (0, 0, 0, 0)),
            pl.BlockSpec((_G, 1, 1), lambda i: (0, 0, 0)),
            pl.BlockSpec((4, _RA, 128), lambda i: (0, i, 0)),
            pl.BlockSpec((4, _RA, 128), lambda i: (0, i, 0)),
        ],
        out_specs=[
            pl.BlockSpec((_RA * 128,), lambda i: (i,)),
            pl.BlockSpec((1, 1), lambda i: (0, 0)),
            pl.BlockSpec((1, 1), lambda i: (0, 0)),
        ],
        out_shape=[
            jax.ShapeDtypeStruct((_P,), jnp.int32),
            jax.ShapeDtypeStruct((1, 1), jnp.float32),
            jax.ShapeDtypeStruct((1, 1), jnp.float32),
        ],
    )(bto, bti, bpi, tt, lab, pcft, loct)

    ce = pl.pallas_call(
        _conf_body,
        grid=(_NBC,),
        in_specs=[
            pl.BlockSpec((_BC, _C), lambda i: (i, 0)),
            pl.BlockSpec((_BC,), lambda i: (i,)),
        ],
        out_specs=pl.BlockSpec((_BC,), lambda i: (i,)),
        out_shape=jax.ShapeDtypeStruct((_P,), jnp.float32),
    )(conf_flat, conf_t)

    out = pl.pallas_call(
        _final_body,
        in_specs=[
            pl.BlockSpec((_P,), lambda: (0,)),
            pl.BlockSpec((_P,), lambda: (0,)),
            pl.BlockSpec((1, 1), lambda: (0, 0)),
            pl.BlockSpec((1, 1), lambda: (0, 0)),
        ],
        out_specs=pl.BlockSpec((1, 1), lambda: (0, 0)),
        out_shape=jax.ShapeDtypeStruct((1, 1), jnp.float32),
    )(ce, conf_t, locl, npos)

    return out[0, 0]


# speedup vs baseline: 4.2935x; 1.0383x over previous
import jax
import jax.numpy as jnp
from jax import lax
from jax.experimental import pallas as pl
from jax.experimental.pallas import tpu as pltpu

_P = 131072
_G = 50
_C = 81
_R = _P // 128
_RA = 8
_BC = 2048
_NBA = _R // _RA
_NBC = _P // _BC


def _match_body(pft_ref, tt_ref, bto_ref, bti_ref, accv_ref, acci_ref,
                bpi_ref):
    blk = pl.program_id(0)
    pft = pft_ref[...]
    tt = tt_ref[...]
    px1, py1 = pft[0][None], pft[1][None]
    px2, py2 = pft[2][None], pft[3][None]
    tx1, ty1, tx2, ty2 = tt[0], tt[1], tt[2], tt[3]
    w = jnp.maximum(jnp.minimum(px2, tx2) - jnp.maximum(px1, tx1), 0.0)
    h = jnp.maximum(jnp.minimum(py2, ty2) - jnp.maximum(py1, ty1), 0.0)
    inter = w * h
    aa = (px2 - px1) * (py2 - py1)
    ab = (tx2 - tx1) * (ty2 - ty1)
    iou = inter / (aa + ab - inter)

    g_iota = lax.broadcasted_iota(jnp.int32, (_G, _RA, 128), 0)
    bto = jnp.max(iou, axis=0)
    bti = jnp.min(jnp.where(iou == bto[None], g_iota, _G), axis=0)
    bto_ref[...] = bto
    bti_ref[...] = bti

    gp = (blk * (_RA * 128)
          + lax.broadcasted_iota(jnp.int32, (_RA, 128), 0) * 128
          + lax.broadcasted_iota(jnp.int32, (_RA, 128), 1))[None]

    @pl.when(blk == 0)
    def _():
        accv_ref[...] = jnp.full((_G, _RA, 128), -1.0, jnp.float32)
        acci_ref[...] = jnp.zeros((_G, _RA, 128), jnp.int32)

    acc_v = accv_ref[...]
    upd = iou > acc_v
    accv_ref[...] = jnp.where(upd, iou, acc_v)
    acci_ref[...] = jnp.where(upd, gp, acci_ref[...])

    @pl.when(blk == _NBA - 1)
    def _():
        av = accv_ref[...]
        vmax = jnp.max(av, axis=(1, 2), keepdims=True)
        bpi_ref[...] = jnp.min(
            jnp.where(av == vmax, acci_ref[...], _P), axis=(1, 2),
            keepdims=True)


def _targets_body(bto_ref, bti_ref, bpi_ref, tt_ref, lab_ref, pcft_ref,
                  loct_ref, ct_ref, locl_ref, npos_ref):
    blk = pl.program_id(0)
    bto = bto_ref[...]
    bti = bti_ref[...]
    bpi = bpi_ref[...]
    gp = (blk * (_RA * 128)
          + lax.broadcasted_iota(jnp.int32, (_RA, 128), 0) * 128
          + lax.broadcasted_iota(jnp.int32, (_RA, 128), 1))
    g_iota = lax.broadcasted_iota(jnp.int32, (_G, _RA, 128), 0)
    forced_g = jnp.max(jnp.where(bpi == gp[None], g_iota, -1), axis=0)
    forced = forced_g >= 0
    bti2 = jnp.where(forced, forced_g, bti)
    bto2 = jnp.where(forced, 2.0, bto)
    onehot = g_iota == bti2[None]

    lab = lab_ref[...]
    labm = jnp.sum(jnp.where(onehot, lab, 0.0), axis=0)
    conf_t = jnp.where(bto2 >= 0.5, labm.astype(jnp.int32) + 1, 0)

    tt = tt_ref[...]
    mx1 = jnp.sum(jnp.where(onehot, tt[0], 0.0), axis=0)
    my1 = jnp.sum(jnp.where(onehot, tt[1], 0.0), axis=0)
    mx2 = jnp.sum(jnp.where(onehot, tt[2], 0.0), axis=0)
    my2 = jnp.sum(jnp.where(onehot, tt[3], 0.0), axis=0)

    pcf = pcft_ref[...]
    cx, cy, w, h = pcf[0], pcf[1], pcf[2], pcf[3]
    gcx = ((mx1 + mx2) * 0.5 - cx) / (0.1 * w)
    gcy = ((my1 + my2) * 0.5 - cy) / (0.1 * h)
    gw = jnp.log((mx2 - mx1) / w) / 0.2
    gh = jnp.log((my2 - my1) / h) / 0.2

    loc = loct_ref[...]

    def sl1(d):
        a = jnp.abs(d)
        return jnp.where(a < 1.0, 0.5 * d * d, a - 0.5)

    l = sl1(loc[0] - gcx) + sl1(loc[1] - gcy) + sl1(loc[2] - gw) \
        + sl1(loc[3] - gh)
    posf = (conf_t > 0).astype(jnp.float32)
    ct_ref[...] = conf_t.reshape(_RA * 128)

    @pl.when(blk == 0)
    def _():
        locl_ref[...] = jnp.zeros((1, 1), jnp.float32)
        npos_ref[...] = jnp.zeros((1, 1), jnp.float32)

    locl_ref[...] = locl_ref[...] + jnp.sum(l * posf)
    npos_ref[...] = npos_ref[...] + jnp.sum(posf)


def _conf_body(x_ref, ct_ref, cl_ref, cepos_ref):
    blk = pl.program_id(0)
    x = x_ref[...]
    ct = ct_ref[...][:, None]
    rmax = jnp.max(x, axis=1, keepdims=True)
    s = jnp.sum(jnp.exp(x - rmax), axis=1, keepdims=True)
    lse = jnp.log(s) + rmax
    cidx = lax.broadcasted_iota(jnp.int32, (_BC, _C), 1)
    xt = jnp.sum(jnp.where(cidx == ct, x, 0.0), axis=1, keepdims=True)
    ce = lse - xt
    posf = (ct > 0).astype(jnp.float32)
    cl = ce * (1.0 - posf)

    @pl.when(blk == 0)
    def _():
        cepos_ref[...] = jnp.zeros((1, 1), jnp.float32)

    cepos_ref[...] = cepos_ref[...] + jnp.sum(ce * posf)
    cl_ref[...] = jnp.swapaxes(cl, 0, 1)[None]


def _final_body(cl_ref, locl_ref, npos_ref, cepos_ref, out_ref):
    cl = cl_ref[...]
    ce_pos = cepos_ref[0, 0]
    bits = lax.bitcast_convert_type(cl, jnp.int32)
    npos = npos_ref[0, 0]
    k = jnp.minimum(3 * npos.astype(jnp.int32), _P - 1)
    kk = k + 1

    def body1(i, acc):
        cand = acc | (jnp.int32(1) << (30 - i))
        cnt = jnp.sum((bits >= cand).astype(jnp.int32))
        return jnp.where(cnt >= kk, cand, acc)

    acc0 = body1(0, jnp.int32(0))

    def body2(i, acc):
        lo = 28 - 2 * i
        c1 = acc | (jnp.int32(1) << lo)
        c2 = acc | (jnp.int32(2) << lo)
        c3 = acc | (jnp.int32(3) << lo)
        n1 = jnp.sum((bits >= c1).astype(jnp.int32))
        n2 = jnp.sum((bits >= c2).astype(jnp.int32))
        n3 = jnp.sum((bits >= c3).astype(jnp.int32))
        acc = jnp.where(n3 >= kk, c3,
                        jnp.where(n2 >= kk, c2,
                                  jnp.where(n1 >= kk, c1, acc)))
        return acc

    pivot = lax.bitcast_convert_type(lax.fori_loop(0, 15, body2, acc0),
                                     jnp.float32)
    ce_neg = jnp.sum(jnp.where(cl > pivot, cl, 0.0))
    loss = (locl_ref[0, 0] + ce_pos + ce_neg) / npos
    out_ref[...] = jnp.zeros((1, 1), jnp.float32) + loss


def kernel(loc_data, conf_data, priors, ground_truth):
    pft = priors[0].reshape(_P, 4).T.reshape(4, _R, 128)
    pcft = priors[1].reshape(_P, 4).T.reshape(4, _R, 128)
    tt = ground_truth[:, :4].T.reshape(4, _G, 1, 1)
    lab = ground_truth[:, 4].reshape(_G, 1, 1)
    conf_flat = conf_data.reshape(_P, _C)
    loct = loc_data.reshape(_P, 4).T.reshape(4, _R, 128)

    bto, bti, _, _, bpi = pl.pallas_call(
        _match_body,
        grid=(_NBA,),
        in_specs=[
            pl.BlockSpec((4, _RA, 128), lambda i: (0, i, 0)),
            pl.BlockSpec((4, _G, 1, 1), lambda i: (0, 0, 0, 0)),
        ],
        out_specs=[
            pl.BlockSpec((_RA, 128), lambda i: (i, 0)),
            pl.BlockSpec((_RA, 128), lambda i: (i, 0)),
            pl.BlockSpec((_G, _RA, 128), lambda i: (0, 0, 0)),
            pl.BlockSpec((_G, _RA, 128), lambda i: (0, 0, 0)),
            pl.BlockSpec((_G, 1, 1), lambda i: (0, 0, 0)),
        ],
        out_shape=[
            jax.ShapeDtypeStruct((_R, 128), jnp.float32),
            jax.ShapeDtypeStruct((_R, 128), jnp.int32),
            jax.ShapeDtypeStruct((_G, _RA, 128), jnp.float32),
            jax.ShapeDtypeStruct((_G, _RA, 128), jnp.int32),
            jax.ShapeDtypeStruct((_G, 1, 1), jnp.int32),
        ],
    )(pft, tt)

    conf_t, locl, npos = pl.pallas_call(
        _targets_body,
        grid=(_NBA,),
        in_specs=[
            pl.BlockSpec((_RA, 128), lambda i: (i, 0)),
            pl.BlockSpec((_RA, 128), lambda i: (i, 0)),
            pl.BlockSpec((_G, 1, 1), lambda i: (0, 0, 0)),
            pl.BlockSpec((4, _G, 1, 1), lambda i: (0, 0, 0, 0)),
            pl.BlockSpec((_G, 1, 1), lambda i: (0, 0, 0)),
            pl.BlockSpec((4, _RA, 128), lambda i: (0, i, 0)),
            pl.BlockSpec((4, _RA, 128), lambda i: (0, i, 0)),
        ],
        out_specs=[
            pl.BlockSpec((_RA * 128,), lambda i: (i,)),
            pl.BlockSpec((1, 1), lambda i: (0, 0)),
            pl.BlockSpec((1, 1), lambda i: (0, 0)),
        ],
        out_shape=[
            jax.ShapeDtypeStruct((_P,), jnp.int32),
            jax.ShapeDtypeStruct((1, 1), jnp.float32),
            jax.ShapeDtypeStruct((1, 1), jnp.float32),
        ],
    )(bto, bti, bpi, tt, lab, pcft, loct)

    cl3, cepos = pl.pallas_call(
        _conf_body,
        grid=(_NBC,),
        in_specs=[
            pl.BlockSpec((_BC, _C), lambda i: (i, 0)),
            pl.BlockSpec((_BC,), lambda i: (i,)),
        ],
        out_specs=[
            pl.BlockSpec((1, 1, _BC), lambda i: (i, 0, 0)),
            pl.BlockSpec((1, 1), lambda i: (0, 0)),
        ],
        out_shape=[
            jax.ShapeDtypeStruct((_NBC, 1, _BC), jnp.float32),
            jax.ShapeDtypeStruct((1, 1), jnp.float32),
        ],
    )(conf_flat, conf_t)

    out = pl.pallas_call(
        _final_body,
        in_specs=[
            pl.BlockSpec((_NBC, 1, _BC), lambda: (0, 0, 0)),
            pl.BlockSpec((1, 1), lambda: (0, 0)),
            pl.BlockSpec((1, 1), lambda: (0, 0)),
            pl.BlockSpec((1, 1), lambda: (0, 0)),
        ],
        out_specs=pl.BlockSpec((1, 1), lambda: (0, 0)),
        out_shape=jax.ShapeDtypeStruct((1, 1), jnp.float32),
    )(cl3, locl, npos, cepos)

    return out[0, 0]
